# Initial kernel scaffold; baseline (speedup 1.0000x reference)
import jax
import jax.numpy as jnp
from jax.experimental import pallas as pl


def kernel(token_ids, weight):
    raise NotImplementedError("write your pallas kernel here")


# speedup vs baseline: 1.6039x; 1.6039x over previous
import functools

import jax
import jax.numpy as jnp
from jax import lax
from jax.experimental import pallas as pl
from jax.experimental.pallas import tpu as pltpu
from jax.experimental.pallas import tpu_sc as plsc

_NUM_CORES = 2
_NUM_SUBCORES = 16
_NUM_WORKERS = _NUM_CORES * _NUM_SUBCORES
_CHUNK = 512


def _gather_rows(wpad, flat_ids):
    num_indices = flat_ids.shape[0]
    dim = wpad.shape[1]
    per_worker = num_indices // _NUM_WORKERS
    mesh = plsc.VectorSubcoreMesh(core_axis_name="c", subcore_axis_name="s")

    @functools.partial(
        pl.kernel,
        mesh=mesh,
        out_type=jax.ShapeDtypeStruct((num_indices, dim), wpad.dtype),
        scratch_types=[
            pltpu.VMEM((_CHUNK,), jnp.int32),
            pltpu.VMEM((_CHUNK, dim), wpad.dtype),
            pltpu.SemaphoreType.DMA,
        ],
    )
    def gather_kernel(table_hbm, idx_hbm, out_hbm, idx_v, rows_v, sem):
        wid = lax.axis_index("s") * _NUM_CORES + lax.axis_index("c")
        base = wid * per_worker

        @pl.loop(0, per_worker, step=_CHUNK)
        def _(off):
            pltpu.sync_copy(idx_hbm.at[pl.ds(base + off, _CHUNK)], idx_v)
            pltpu.async_copy(table_hbm.at[idx_v], rows_v, sem).wait()
            pltpu.sync_copy(rows_v, out_hbm.at[pl.ds(base + off, _CHUNK)])

    return gather_kernel(wpad, flat_ids)


def kernel(token_ids, weight):
    batch, seq = token_ids.shape
    num_rows, dim = weight.shape
    flat_ids = token_ids.reshape(batch * seq)
    wpad = jnp.concatenate(
        [weight, jnp.zeros((num_rows, 128 - dim), weight.dtype)], axis=1
    )
    out = _gather_rows(wpad, flat_ids)
    return out[:, :dim].reshape(batch, seq, dim)


# speedup vs baseline: 2.1397x; 1.3341x over previous
import functools

import jax
import jax.numpy as jnp
from jax import lax
from jax.experimental import pallas as pl
from jax.experimental.pallas import tpu as pltpu
from jax.experimental.pallas import tpu_sc as plsc

_NUM_CORES = 2
_NUM_SUBCORES = 16
_NUM_WORKERS = _NUM_CORES * _NUM_SUBCORES
_ROWS_PER_CHUNK = 8


def _gather_rows(wpad, flat_ids, batch, seq):
    pad_dim = wpad.shape[1]
    rows_per_worker = batch // _NUM_WORKERS
    chunk = _ROWS_PER_CHUNK * seq
    mesh = plsc.VectorSubcoreMesh(core_axis_name="c", subcore_axis_name="s")

    @functools.partial(
        pl.kernel,
        mesh=mesh,
        out_type=jax.ShapeDtypeStruct((batch, seq, pad_dim), wpad.dtype),
        scratch_types=[
            pltpu.VMEM((chunk,), jnp.int32),
            pltpu.VMEM((chunk, pad_dim), wpad.dtype),
            pltpu.SemaphoreType.DMA,
        ],
    )
    def gather_kernel(table_hbm, idx_hbm, out_hbm, idx_v, rows_v, sem):
        wid = lax.axis_index("s") * _NUM_CORES + lax.axis_index("c")
        row0 = wid * rows_per_worker

        @pl.loop(0, rows_per_worker, step=_ROWS_PER_CHUNK)
        def _(r):
            pltpu.sync_copy(idx_hbm.at[pl.ds((row0 + r) * seq, chunk)], idx_v)
            pltpu.async_copy(table_hbm.at[idx_v], rows_v, sem).wait()
            for j in range(_ROWS_PER_CHUNK):
                pltpu.sync_copy(
                    rows_v.at[pl.ds(j * seq, seq)],
                    out_hbm.at[row0 + r + j],
                )

    return gather_kernel(wpad, flat_ids)


def kernel(token_ids, weight):
    batch, seq = token_ids.shape
    num_rows, dim = weight.shape
    flat_ids = token_ids.reshape(batch * seq)
    wpad = jnp.concatenate(
        [weight, jnp.zeros((num_rows, 128 - dim), weight.dtype)], axis=1
    )
    out_pad = _gather_rows(wpad, flat_ids, batch, seq)
    return out_pad[:, :, :dim]
